# baseline (speedup 1.0000x reference)
import jax
import jax.numpy as jnp
from jax.experimental import pallas as pl

_TOD_SIZE = 288
_DOW_SIZE = 7


def _emb_block_kernel(pack_ref, node_ref, tod_ref, dow_ref, out_ref):
    n = pack_ref.shape[-1]
    pidx = pack_ref[0, 0, 0, :]
    tod_idx = jnp.bitwise_and(pidx, 511)
    dow_idx = jax.lax.shift_right_logical(pidx, 9)

    out_ref[0, 0] = node_ref[...]


def kernel(x, node_emb, time_of_day_emb, day_of_week_emb):
    B, T, N, _ = x.shape
    D_node = node_emb.shape[1]
    D_tod = time_of_day_emb.shape[1]
    D_dow = day_of_week_emb.shape[1]
    D = D_dow + D_tod + D_node
    BT = B * T

    tod_idx = (x[:, :, :, 1] * float(_TOD_SIZE)).astype(jnp.int32)
    dow_idx = (x[:, :, :, 2] * float(_DOW_SIZE)).astype(jnp.int32)
    pack = (tod_idx + (dow_idx << 9)).reshape(B, T, 1, N)

    tod96 = (jnp.zeros((_TOD_SIZE, D), jnp.float32)
             .at[:, D_dow:D_dow + D_tod].set(time_of_day_emb)
             .astype(jnp.bfloat16))
    dow96 = (jnp.zeros((8, D), jnp.float32)
             .at[:_DOW_SIZE, :D_dow].set(day_of_week_emb)
             .astype(jnp.bfloat16))
    node_wide = (jnp.zeros((N, D), jnp.float32)
                 .at[:, D_dow + D_tod:].set(node_emb))

    S = 4
    out = pl.pallas_call(
        _emb_block_kernel,
        grid=(B, T, S),
        in_specs=[
            pl.BlockSpec((1, 1, 1, N // S), lambda i, j, k: (i, j, 0, k)),
            pl.BlockSpec((N // S, D), lambda i, j, k: (k, 0)),
            pl.BlockSpec((_TOD_SIZE, D), lambda i, j, k: (0, 0)),
            pl.BlockSpec((8, D), lambda i, j, k: (0, 0)),
        ],
        out_specs=pl.BlockSpec((1, 1, N // S, D), lambda i, j, k: (i, j, k, 0)),
        out_shape=jax.ShapeDtypeStruct((B, T, N, D), jnp.float32),
    )(pack, node_wide, tod96, dow96)

    return out


# speedup vs baseline: 2.1002x; 2.1002x over previous
import jax
import jax.numpy as jnp
from jax.experimental import pallas as pl

_TOD_SIZE = 288
_DOW_SIZE = 7
_KA = 256
_KB = 40


def _emb_block_kernel(pack_ref, node_ref, taba_ref, tabbd_ref, out_ref):
    r_steps = pack_ref.shape[1]
    n = pack_ref.shape[-1]
    for r in range(r_steps):
        pidx = pack_ref[0, r, 0, :]
        tod_idx = jnp.bitwise_and(pidx, 511)
        dow_idx = jax.lax.shift_right_logical(pidx, 9)
        tod_col = tod_idx[:, None]
        dow_col = dow_idx[:, None]

        iota_a = jax.lax.broadcasted_iota(jnp.int32, (n, _KA), 1)
        oh_a = (tod_col == iota_a).astype(jnp.bfloat16)
        iota_b = jax.lax.broadcasted_iota(jnp.int32, (n, _KB), 1)
        oh_bd = ((tod_col == iota_b + _KA) | (dow_col == iota_b - 32)
                 ).astype(jnp.bfloat16)

        mm = (jnp.dot(oh_a, taba_ref[...], preferred_element_type=jnp.float32)
              + jnp.dot(oh_bd, tabbd_ref[...],
                        preferred_element_type=jnp.float32))
        out_ref[0, r] = mm + node_ref[...]


def kernel(x, node_emb, time_of_day_emb, day_of_week_emb):
    B, T, N, _ = x.shape
    D_node = node_emb.shape[1]
    D_tod = time_of_day_emb.shape[1]
    D_dow = day_of_week_emb.shape[1]
    D = D_dow + D_tod + D_node

    tod_idx = (x[:, :, :, 1] * float(_TOD_SIZE)).astype(jnp.int32)
    dow_idx = (x[:, :, :, 2] * float(_DOW_SIZE)).astype(jnp.int32)
    pack = (tod_idx + (dow_idx << 9)).reshape(B, T, 1, N)

    taba = (jnp.zeros((_KA, D), jnp.float32)
            .at[:, D_dow:D_dow + D_tod].set(time_of_day_emb[:_KA])
            .astype(jnp.bfloat16))
    tabbd = (jnp.zeros((_KB, D), jnp.float32)
             .at[:_TOD_SIZE - _KA, D_dow:D_dow + D_tod]
             .set(time_of_day_emb[_KA:])
             .at[32:32 + _DOW_SIZE, :D_dow].set(day_of_week_emb)
             .astype(jnp.bfloat16))
    node_wide = (jnp.zeros((N, D), jnp.float32)
                 .at[:, D_dow + D_tod:].set(node_emb))

    R = 6
    out = pl.pallas_call(
        _emb_block_kernel,
        grid=(B, T // R),
        in_specs=[
            pl.BlockSpec((1, R, 1, N), lambda i, j: (i, j, 0, 0)),
            pl.BlockSpec((N, D), lambda i, j: (0, 0)),
            pl.BlockSpec((_KA, D), lambda i, j: (0, 0)),
            pl.BlockSpec((_KB, D), lambda i, j: (0, 0)),
        ],
        out_specs=pl.BlockSpec((1, R, N, D), lambda i, j: (i, j, 0, 0)),
        out_shape=jax.ShapeDtypeStruct((B, T, N, D), jnp.float32),
    )(pack, node_wide, taba, tabbd)

    return out


# speedup vs baseline: 2.1878x; 1.0417x over previous
import jax
import jax.numpy as jnp
from jax.experimental import pallas as pl

_TOD_SIZE = 288
_DOW_SIZE = 7
_KA = 256
_KB = 40


def _emb_block_kernel(pack_ref, node_ref, taba_ref, tabbd_ref, out_ref):
    r_steps = pack_ref.shape[1]
    n = pack_ref.shape[-1]
    for r in range(r_steps):
        pidx = pack_ref[0, r, 0, :]
        tod_idx = jnp.bitwise_and(pidx, 511)
        dow_idx = jax.lax.shift_right_logical(pidx, 9)
        tod_col = tod_idx[:, None]
        dow_col = dow_idx[:, None]

        iota_a = jax.lax.broadcasted_iota(jnp.int32, (n, _KA), 1)
        oh_a = (tod_col == iota_a).astype(jnp.bfloat16)
        iota_b = jax.lax.broadcasted_iota(jnp.int32, (n, _KB), 1)
        oh_bd = ((tod_col == iota_b + _KA) | (dow_col == iota_b - 32)
                 ).astype(jnp.bfloat16)

        mm = (jnp.dot(oh_a, taba_ref[...], preferred_element_type=jnp.float32)
              + jnp.dot(oh_bd, tabbd_ref[...],
                        preferred_element_type=jnp.float32))
        out_ref[0, r] = mm + node_ref[...]


def kernel(x, node_emb, time_of_day_emb, day_of_week_emb):
    B, T, N, _ = x.shape
    D_node = node_emb.shape[1]
    D_tod = time_of_day_emb.shape[1]
    D_dow = day_of_week_emb.shape[1]
    D = D_dow + D_tod + D_node

    tod_idx = (x[:, :, :, 1] * float(_TOD_SIZE)).astype(jnp.int32)
    dow_idx = (x[:, :, :, 2] * float(_DOW_SIZE)).astype(jnp.int32)
    pack = (tod_idx + (dow_idx << 9)).reshape(B, T, 1, N)

    taba = (jnp.zeros((_KA, D), jnp.float32)
            .at[:, D_dow:D_dow + D_tod].set(time_of_day_emb[:_KA])
            .astype(jnp.bfloat16))
    tabbd = (jnp.zeros((_KB, D), jnp.float32)
             .at[:_TOD_SIZE - _KA, D_dow:D_dow + D_tod]
             .set(time_of_day_emb[_KA:])
             .at[32:32 + _DOW_SIZE, :D_dow].set(day_of_week_emb)
             .astype(jnp.bfloat16))
    node_wide = (jnp.zeros((N, D), jnp.float32)
                 .at[:, D_dow + D_tod:].set(node_emb))

    R = 12
    out = pl.pallas_call(
        _emb_block_kernel,
        grid=(B, T // R),
        in_specs=[
            pl.BlockSpec((1, R, 1, N), lambda i, j: (i, j, 0, 0)),
            pl.BlockSpec((N, D), lambda i, j: (0, 0)),
            pl.BlockSpec((_KA, D), lambda i, j: (0, 0)),
            pl.BlockSpec((_KB, D), lambda i, j: (0, 0)),
        ],
        out_specs=pl.BlockSpec((1, R, N, D), lambda i, j: (i, j, 0, 0)),
        out_shape=jax.ShapeDtypeStruct((B, T, N, D), jnp.float32),
    )(pack, node_wide, taba, tabbd)

    return out


# speedup vs baseline: 2.2292x; 1.0189x over previous
import jax
import jax.numpy as jnp
from jax.experimental import pallas as pl

_TOD_SIZE = 288
_DOW_SIZE = 7
_KA = 256
_KB = 40


def _emb_block_kernel(pack_ref, node_ref, taba_ref, tabbd_ref, out_ref):
    b_steps = pack_ref.shape[0]
    r_steps = pack_ref.shape[1]
    n = pack_ref.shape[-1]
    for b in range(b_steps):
      for r in range(r_steps):
        pidx = pack_ref[b, r, 0, :]
        tod_idx = jnp.bitwise_and(pidx, 511)
        dow_idx = jax.lax.shift_right_logical(pidx, 9)
        tod_col = tod_idx[:, None]
        dow_col = dow_idx[:, None]

        iota_a = jax.lax.broadcasted_iota(jnp.int32, (n, _KA), 1)
        oh_a = (tod_col == iota_a).astype(jnp.bfloat16)
        iota_b = jax.lax.broadcasted_iota(jnp.int32, (n, _KB), 1)
        oh_bd = ((tod_col == iota_b + _KA) | (dow_col == iota_b - 32)
                 ).astype(jnp.bfloat16)

        mm = (jnp.dot(oh_a, taba_ref[...], preferred_element_type=jnp.float32)
              + jnp.dot(oh_bd, tabbd_ref[...],
                        preferred_element_type=jnp.float32))
        out_ref[b, r] = mm + node_ref[...]


def kernel(x, node_emb, time_of_day_emb, day_of_week_emb):
    B, T, N, _ = x.shape
    D_node = node_emb.shape[1]
    D_tod = time_of_day_emb.shape[1]
    D_dow = day_of_week_emb.shape[1]
    D = D_dow + D_tod + D_node

    tod_idx = (x[:, :, :, 1] * float(_TOD_SIZE)).astype(jnp.int32)
    dow_idx = (x[:, :, :, 2] * float(_DOW_SIZE)).astype(jnp.int32)
    pack = (tod_idx + (dow_idx << 9)).reshape(B, T, 1, N)

    taba = (jnp.zeros((_KA, D), jnp.float32)
            .at[:, D_dow:D_dow + D_tod].set(time_of_day_emb[:_KA])
            .astype(jnp.bfloat16))
    tabbd = (jnp.zeros((_KB, D), jnp.float32)
             .at[:_TOD_SIZE - _KA, D_dow:D_dow + D_tod]
             .set(time_of_day_emb[_KA:])
             .at[32:32 + _DOW_SIZE, :D_dow].set(day_of_week_emb)
             .astype(jnp.bfloat16))
    node_wide = (jnp.zeros((N, D), jnp.float32)
                 .at[:, D_dow + D_tod:].set(node_emb))

    R = 12
    out = pl.pallas_call(
        _emb_block_kernel,
        grid=(B // 2, T // R),
        in_specs=[
            pl.BlockSpec((2, R, 1, N), lambda i, j: (i, j, 0, 0)),
            pl.BlockSpec((N, D), lambda i, j: (0, 0)),
            pl.BlockSpec((_KA, D), lambda i, j: (0, 0)),
            pl.BlockSpec((_KB, D), lambda i, j: (0, 0)),
        ],
        out_specs=pl.BlockSpec((2, R, N, D), lambda i, j: (i, j, 0, 0)),
        out_shape=jax.ShapeDtypeStruct((B, T, N, D), jnp.float32),
    )(pack, node_wide, taba, tabbd)

    return out
